# baseline (speedup 1.0000x reference)
import functools

import jax
import jax.numpy as jnp
from jax import lax
from jax.experimental import pallas as pl
from jax.experimental.pallas import tpu as pltpu
from jax.experimental.pallas import tpu_sc as plsc

_B, _C, _H, _W = 32, 2, 512, 512
_HW = _H * _W
_NC, _NS, _L = 2, 16, 16
_NWORK = _NC * _NS
_CHUNK = 16384
_NCHUNK = _HW // _CHUNK
_INNER = _CHUNK // _L

_LN2 = 0.6931471805599453
_SCALE = -1.0 / float(_B * _HW)
_LOG_C = (
    9.975032551941786e-06 - 127.0 * _LN2,
    0.9992354838332754,
    -0.4902307234234105,
    0.285272681090574,
    -0.13158182508876004,
    0.03044900453866886,
)

_mesh = plsc.VectorSubcoreMesh(
    core_axis_name="c", subcore_axis_name="s", num_cores=_NC, num_subcores=_NS
)


@functools.partial(
    pl.kernel,
    out_type=jax.ShapeDtypeStruct((_NWORK, _L), jnp.float32),
    mesh=_mesh,
    scratch_types=[
        pltpu.VMEM((2, _CHUNK), jnp.float32),
        pltpu.VMEM((2, _CHUNK), jnp.float32),
        pltpu.VMEM((2, _CHUNK), jnp.int32),
        pltpu.VMEM((_L,), jnp.float32),
        pltpu.SemaphoreType.DMA,
        pltpu.SemaphoreType.DMA,
    ],
    compiler_params=pltpu.CompilerParams(needs_layout_passes=False),
)
def _focal_partials(x_hbm, t_hbm, out_hbm, a_buf, b_buf, t_buf, out_v, sem0, sem1):
    cid = lax.axis_index("c")
    sid = lax.axis_index("s")
    w = sid * _NC + cid

    sems = (sem0, sem1)

    def start(slot, ci):
        off = pl.ds(ci * _CHUNK, _CHUNK)
        sem = sems[slot]
        return (
            pltpu.async_copy(x_hbm.at[w, 0, off], a_buf.at[slot], sem),
            pltpu.async_copy(x_hbm.at[w, 1, off], b_buf.at[slot], sem),
            pltpu.async_copy(t_hbm.at[w, off], t_buf.at[slot], sem),
        )

    inflight = {0: start(0, 0)}

    def step(i, acc, slot):
        off = pl.ds(i * _L, _L)
        a = a_buf[slot, off]
        b = b_buf[slot, off]
        t = t_buf[slot, off]
        p = jnp.where(t == 0, a, b)
        bits = plsc.bitcast(p, jnp.int32)
        e = lax.convert_element_type(
            lax.shift_right_arithmetic(bits, 23), jnp.float32
        )
        r = plsc.bitcast(
            lax.bitwise_or(lax.bitwise_and(bits, 0x007FFFFF), 0x3F800000),
            jnp.float32,
        ) - 1.0
        poly = _LOG_C[5]
        for c in (_LOG_C[4], _LOG_C[3], _LOG_C[2], _LOG_C[1], _LOG_C[0]):
            poly = poly * r + c
        logp = poly + e * _LN2
        om = 1.0 - p
        return acc + (om * om) * logp

    acc = jnp.zeros((_L,), jnp.float32)
    for ci in range(_NCHUNK):
        slot = ci % 2
        if ci + 1 < _NCHUNK:
            inflight[1 - slot] = start(1 - slot, ci + 1)
        for cp in inflight[slot]:
            cp.wait()
        acc = lax.fori_loop(
            0, _INNER, lambda i, a: step(i, a, slot), acc, unroll=4
        )

    out_v[...] = acc * _SCALE
    pltpu.sync_copy(out_v, out_hbm.at[w])


def kernel(inputs, targets):
    x = inputs.reshape(_B, _C, _HW)
    t = targets.reshape(_B, _HW)
    partials = _focal_partials(x, t)
    return jnp.sum(partials)


# speedup vs baseline: 1.4143x; 1.4143x over previous
import functools

import jax
import jax.numpy as jnp
from jax import lax
from jax.experimental import pallas as pl
from jax.experimental.pallas import tpu as pltpu
from jax.experimental.pallas import tpu_sc as plsc

_B, _C, _H, _W = 32, 2, 512, 512
_HW = _H * _W
_NC, _NS, _L = 2, 16, 16
_NWORK = _NC * _NS
_K_SC = 4
_WPB = _NWORK // _K_SC
_WPIX = _HW // _WPB
_CHUNK = 16384
_NCHUNK = _WPIX // _CHUNK
_INNER = _CHUNK // _L

_LN2 = 0.6931471805599453
_SCALE = -1.0 / float(_B * _HW)
_LOG_C = (
    9.975032551941786e-06 - 127.0 * _LN2,
    0.9992354838332754,
    -0.4902307234234105,
    0.285272681090574,
    -0.13158182508876004,
    0.03044900453866886,
)

_mesh = plsc.VectorSubcoreMesh(
    core_axis_name="c", subcore_axis_name="s", num_cores=_NC, num_subcores=_NS
)


@functools.partial(
    pl.kernel,
    out_type=jax.ShapeDtypeStruct((_NWORK, _L), jnp.float32),
    mesh=_mesh,
    scratch_types=[
        pltpu.VMEM((2, _CHUNK), jnp.float32),
        pltpu.VMEM((2, _CHUNK), jnp.float32),
        pltpu.VMEM((2, _CHUNK), jnp.int32),
        pltpu.VMEM((_L,), jnp.float32),
        pltpu.SemaphoreType.DMA,
        pltpu.SemaphoreType.DMA,
    ],
    compiler_params=pltpu.CompilerParams(needs_layout_passes=False),
)
def _focal_sc(x_hbm, t_hbm, out_hbm, a_buf, b_buf, t_buf, out_v, sem0, sem1):
    cid = lax.axis_index("c")
    sid = lax.axis_index("s")
    w = sid * _NC + cid
    bb = w // _WPB
    base = (w % _WPB) * _WPIX

    sems = (sem0, sem1)

    def start(slot, ci):
        off = pl.ds(base + ci * _CHUNK, _CHUNK)
        sem = sems[slot]
        return (
            pltpu.async_copy(x_hbm.at[bb, 0, off], a_buf.at[slot], sem),
            pltpu.async_copy(x_hbm.at[bb, 1, off], b_buf.at[slot], sem),
            pltpu.async_copy(t_hbm.at[bb, off], t_buf.at[slot], sem),
        )

    inflight = {0: start(0, 0)}

    def step(i, acc, slot):
        off = pl.ds(i * _L, _L)
        a = a_buf[slot, off]
        b = b_buf[slot, off]
        t = t_buf[slot, off]
        p = jnp.where(t == 0, a, b)
        bits = plsc.bitcast(p, jnp.int32)
        e = lax.convert_element_type(
            lax.shift_right_arithmetic(bits, 23), jnp.float32
        )
        r = plsc.bitcast(
            lax.bitwise_or(lax.bitwise_and(bits, 0x007FFFFF), 0x3F800000),
            jnp.float32,
        ) - 1.0
        poly = _LOG_C[5]
        for c in (_LOG_C[4], _LOG_C[3], _LOG_C[2], _LOG_C[1], _LOG_C[0]):
            poly = poly * r + c
        logp = poly + e * _LN2
        om = 1.0 - p
        return acc + (om * om) * logp

    acc = jnp.zeros((_L,), jnp.float32)
    for ci in range(_NCHUNK):
        slot = ci % 2
        if ci + 1 < _NCHUNK:
            inflight[1 - slot] = start(1 - slot, ci + 1)
        for cp in inflight[slot]:
            cp.wait()
        acc = lax.fori_loop(
            0, _INNER, lambda i, a: step(i, a, slot), acc, unroll=4
        )

    out_v[...] = acc
    pltpu.sync_copy(out_v, out_hbm.at[w])


def _tc_body(x_ref, t_ref, o_ref):
    a = x_ref[0, 0]
    b = x_ref[0, 1]
    t = t_ref[0]
    p = jnp.where(t == 0, a, b)
    bits = lax.bitcast_convert_type(p, jnp.int32)
    e = lax.convert_element_type(
        lax.shift_right_arithmetic(bits, 23), jnp.float32
    )
    r = lax.bitcast_convert_type(
        lax.bitwise_or(lax.bitwise_and(bits, 0x007FFFFF), 0x3F800000),
        jnp.float32,
    ) - 1.0
    poly = jnp.float32(_LOG_C[5])
    for c in (_LOG_C[4], _LOG_C[3], _LOG_C[2], _LOG_C[1], _LOG_C[0]):
        poly = poly * r + jnp.float32(c)
    logp = poly + e * jnp.float32(_LN2)
    om = 1.0 - p
    vals = (om * om) * logp
    o_ref[0] = jnp.sum(vals.reshape(256, 8, 128), axis=0)


_N_TC = _B - _K_SC

_focal_tc = pl.pallas_call(
    _tc_body,
    grid=(_N_TC,),
    in_specs=[
        pl.BlockSpec((1, _C, 2048, 128), lambda i: (i + _K_SC, 0, 0, 0)),
        pl.BlockSpec((1, 2048, 128), lambda i: (i + _K_SC, 0, 0)),
    ],
    out_specs=pl.BlockSpec((1, 8, 128), lambda i: (i, 0, 0)),
    out_shape=jax.ShapeDtypeStruct((_N_TC, 8, 128), jnp.float32),
    compiler_params=pltpu.CompilerParams(
        dimension_semantics=("arbitrary",)
    ),
)


def kernel(inputs, targets):
    x_sc = inputs[:_K_SC].reshape(_K_SC, _C, _HW)
    t_sc = targets[:_K_SC].reshape(_K_SC, _HW)
    sc_part = _focal_sc(x_sc, t_sc)
    x_tc = inputs.reshape(_B, _C, 2048, 128)
    t_tc = targets.reshape(_B, 2048, 128)
    tc_part = _focal_tc(x_tc, t_tc)
    return (jnp.sum(sc_part) + jnp.sum(tc_part)) * jnp.float32(_SCALE)


# speedup vs baseline: 2.6621x; 1.8823x over previous
import functools

import jax
import jax.numpy as jnp
from jax import lax
from jax.experimental import pallas as pl
from jax.experimental.pallas import tpu as pltpu
from jax.experimental.pallas import tpu_sc as plsc

_B, _C, _H, _W = 32, 2, 512, 512
_HW = _H * _W
_NC, _NS, _L = 2, 16, 16
_NWORK = _NC * _NS
_K_SC = 4
_WPB = _NWORK // _K_SC
_WPIX = _HW // _WPB
_CHUNK = 16384
_NCHUNK = _WPIX // _CHUNK
_INNER = _CHUNK // _L

_LN2 = 0.6931471805599453
_SCALE = -1.0 / float(_B * _HW)
_LOG_C = (
    9.975032551941786e-06 - 127.0 * _LN2,
    0.9992354838332754,
    -0.4902307234234105,
    0.285272681090574,
    -0.13158182508876004,
    0.03044900453866886,
)

_mesh = plsc.VectorSubcoreMesh(
    core_axis_name="c", subcore_axis_name="s", num_cores=_NC, num_subcores=_NS
)


@functools.partial(
    pl.kernel,
    out_type=jax.ShapeDtypeStruct((_NWORK, _L), jnp.float32),
    mesh=_mesh,
    scratch_types=[
        pltpu.VMEM((2, _CHUNK), jnp.float32),
        pltpu.VMEM((2, _CHUNK), jnp.float32),
        pltpu.VMEM((2, _CHUNK), jnp.int32),
        pltpu.VMEM((_L,), jnp.float32),
        pltpu.SemaphoreType.DMA,
        pltpu.SemaphoreType.DMA,
    ],
    compiler_params=pltpu.CompilerParams(needs_layout_passes=False),
)
def _focal_sc(x_hbm, t_hbm, out_hbm, a_buf, b_buf, t_buf, out_v, sem0, sem1):
    cid = lax.axis_index("c")
    sid = lax.axis_index("s")
    w = sid * _NC + cid
    bb = w // _WPB
    base = (w % _WPB) * _WPIX

    sems = (sem0, sem1)

    def start(slot, ci):
        off = pl.ds(base + ci * _CHUNK, _CHUNK)
        sem = sems[slot]
        return (
            pltpu.async_copy(x_hbm.at[bb, 0, off], a_buf.at[slot], sem),
            pltpu.async_copy(x_hbm.at[bb, 1, off], b_buf.at[slot], sem),
            pltpu.async_copy(t_hbm.at[bb, off], t_buf.at[slot], sem),
        )

    inflight = {0: start(0, 0)}

    def step(i, acc, slot):
        off = pl.ds(i * _L, _L)
        a = a_buf[slot, off]
        b = b_buf[slot, off]
        t = t_buf[slot, off]
        p = jnp.where(t == 0, a, b)
        bits = plsc.bitcast(p, jnp.int32)
        e = lax.convert_element_type(
            lax.shift_right_arithmetic(bits, 23), jnp.float32
        )
        r = plsc.bitcast(
            lax.bitwise_or(lax.bitwise_and(bits, 0x007FFFFF), 0x3F800000),
            jnp.float32,
        ) - 1.0
        poly = _LOG_C[5]
        for c in (_LOG_C[4], _LOG_C[3], _LOG_C[2], _LOG_C[1], _LOG_C[0]):
            poly = poly * r + c
        logp = poly + e * _LN2
        om = 1.0 - p
        return acc + (om * om) * logp

    acc = jnp.zeros((_L,), jnp.float32)
    for ci in range(_NCHUNK):
        slot = ci % 2
        if ci + 1 < _NCHUNK:
            inflight[1 - slot] = start(1 - slot, ci + 1)
        for cp in inflight[slot]:
            cp.wait()
        acc = lax.fori_loop(
            0, _INNER, lambda i, a: step(i, a, slot), acc, unroll=4
        )

    out_v[...] = acc
    pltpu.sync_copy(out_v, out_hbm.at[w])


def _tc_body(x_ref, t_ref, o_ref):
    a = x_ref[0, 0]
    b = x_ref[0, 1]
    t = t_ref[0]
    p = jnp.where(t == 0, a, b)
    bits = lax.bitcast_convert_type(p, jnp.int32)
    e = lax.convert_element_type(
        lax.shift_right_arithmetic(bits, 23), jnp.float32
    )
    r = lax.bitcast_convert_type(
        lax.bitwise_or(lax.bitwise_and(bits, 0x007FFFFF), 0x3F800000),
        jnp.float32,
    ) - 1.0
    poly = jnp.float32(_LOG_C[5])
    for c in (_LOG_C[4], _LOG_C[3], _LOG_C[2], _LOG_C[1], _LOG_C[0]):
        poly = poly * r + jnp.float32(c)
    logp = poly + e * jnp.float32(_LN2)
    om = 1.0 - p
    vals = (om * om) * logp
    s = (
        vals[:, 0:128] + vals[:, 128:256] + vals[:, 256:384] + vals[:, 384:512]
    )
    o_ref[0] = jnp.sum(s.reshape(64, 8, 128), axis=0)


_N_TC = _B - _K_SC

_focal_tc = pl.pallas_call(
    _tc_body,
    grid=(_N_TC,),
    in_specs=[
        pl.BlockSpec((1, _C, _H, _W), lambda i: (i + _K_SC, 0, 0, 0)),
        pl.BlockSpec((1, _H, _W), lambda i: (i + _K_SC, 0, 0)),
    ],
    out_specs=pl.BlockSpec((1, 8, 128), lambda i: (i, 0, 0)),
    out_shape=jax.ShapeDtypeStruct((_N_TC, 8, 128), jnp.float32),
    compiler_params=pltpu.CompilerParams(
        dimension_semantics=("arbitrary",)
    ),
)


def kernel(inputs, targets):
    x_sc = inputs[:_K_SC].reshape(_K_SC, _C, _HW)
    t_sc = targets[:_K_SC].reshape(_K_SC, _HW)
    sc_part = _focal_sc(x_sc, t_sc)
    t4 = targets.reshape(_B, _H, _W)
    tc_part = _focal_tc(inputs, t4)
    return (jnp.sum(sc_part) + jnp.sum(tc_part)) * jnp.float32(_SCALE)
